# baseline (speedup 1.0000x reference)
import functools

import jax
import jax.numpy as jnp
from jax import lax
from jax.experimental import pallas as pl
from jax.experimental.pallas import tpu as pltpu
from jax.experimental.pallas import tpu_sc as plsc

_L = 16


def _sc_embed(idx_hbm, table_hbm, scales_hbm, out_hbm,
              idx_v, rows_v, out_v, sc_v, sem,
              *, b_per_w, d):
    n_chunks = b_per_w * d // _L
    rows_per_chunk = _L // d

    wid = lax.axis_index("s") * 2 + lax.axis_index("c")
    base = wid * b_per_w

    pltpu.sync_copy(scales_hbm, sc_v)
    pltpu.sync_copy(idx_hbm.at[pl.ds(base, b_per_w)], idx_v)
    pltpu.async_copy(table_hbm.at[idx_v], rows_v, sem).wait()

    s = sc_v[...]
    lane = lax.iota(jnp.int32, _L)
    base_row = lax.shift_right_logical(lane, 3)
    col = lax.bitwise_and(lane, d - 1)

    def body(g, carry):
        ri = base_row + g * rows_per_chunk
        v = plsc.load_gather(rows_v, [ri, col])
        out_v[pl.ds(g * _L, _L)] = v * s
        return carry

    lax.fori_loop(0, n_chunks, body, 0)
    pltpu.sync_copy(out_v, out_hbm.at[pl.ds(base * d, b_per_w * d)])


def kernel(point_id, cffs_scaled, cff_scales):
    b = point_id.shape[0]
    _, d = cffs_scaled.shape
    nw = 32
    b_per_w = b // nw

    idx = point_id.astype(jnp.int32)
    scales16 = jnp.tile(cff_scales, _L // d)

    run = pl.kernel(
        functools.partial(_sc_embed, b_per_w=b_per_w, d=d),
        out_type=jax.ShapeDtypeStruct((b * d,), jnp.float32),
        mesh=plsc.VectorSubcoreMesh(core_axis_name="c", subcore_axis_name="s"),
        compiler_params=pltpu.CompilerParams(
            needs_layout_passes=False, use_tc_tiling_on_sc=False),
        scratch_types=[
            pltpu.VMEM((b_per_w,), jnp.int32),
            pltpu.VMEM((b_per_w, d), jnp.float32),
            pltpu.VMEM((b_per_w * d,), jnp.float32),
            pltpu.VMEM((_L,), jnp.float32),
            pltpu.SemaphoreType.DMA,
        ],
    )
    out = run(idx, cffs_scaled, scales16)
    return out.reshape(b, d)
